# Initial kernel scaffold; baseline (speedup 1.0000x reference)
import jax
import jax.numpy as jnp
from jax.experimental import pallas as pl


def kernel(x):
    raise NotImplementedError("write your pallas kernel here")


# speedup vs baseline: 1.2567x; 1.2567x over previous
import functools

import jax
import jax.numpy as jnp
from jax.experimental import pallas as pl
from jax.experimental.pallas import tpu as pltpu

_VMEM_LIMIT = 64 * 1024 * 1024


def _interp_matrix_t(n_in: int, n_out: int) -> jnp.ndarray:
    if n_out == 1 or n_in == 1:
        src = jnp.zeros((n_out,), dtype=jnp.float32)
    else:
        src = jnp.arange(n_out, dtype=jnp.float32) * ((n_in - 1) / (n_out - 1))
    i0 = jnp.clip(jnp.floor(src).astype(jnp.int32), 0, n_in - 1)
    i1 = jnp.clip(i0 + 1, 0, n_in - 1)
    frac = src - i0.astype(jnp.float32)
    m0 = jax.nn.one_hot(i0, n_in, dtype=jnp.float32) * (1.0 - frac)[:, None]
    m1 = jax.nn.one_hot(i1, n_in, dtype=jnp.float32) * frac[:, None]
    return (m0 + m1).T


def _up2x_kernel(x_ref, awt_ref, o_ref, *, h_in: int):
    bc, h, w = x_ref.shape
    w_out = awt_ref.shape[1]

    u = jnp.dot(
        x_ref[...].reshape(bc * h, w), awt_ref[...],
        preferred_element_type=jnp.float32,
    ).reshape(bc, h, w_out)

    denom = 1.0 / (2 * h_in - 1)
    ki = jax.lax.broadcasted_iota(jnp.int32, (h, w_out), 0)
    k = ki.astype(jnp.float32)
    fe = jnp.where(k == 0, 0.0, 1.0 - k * denom)
    fo = (h_in - 1 - k) * denom

    u_m1 = jnp.concatenate([u[:, :1], u[:, :-1]], axis=1)
    u_p1 = jnp.concatenate([u[:, 1:], u[:, -1:]], axis=1)

    out_even = (1.0 - fe) * u_m1 + fe * u
    out_odd = (1.0 - fo) * u + fo * u_p1

    o_ref[...] = jnp.stack([out_even, out_odd], axis=2).reshape(
        bc, 2 * h, w_out)


def kernel(x: jnp.ndarray) -> jnp.ndarray:
    n, c, h, w = x.shape
    h_out, w_out = 2 * h, 2 * w
    b = n * c

    a_w_t = _interp_matrix_t(w, w_out)

    bc = 64
    bc = max(1, min(bc, b))
    num_blocks = -(-b // bc)
    b_pad = num_blocks * bc

    x_flat = x.reshape(b, h, w)
    if b_pad != b:
        x_flat = jnp.pad(x_flat, ((0, b_pad - b), (0, 0), (0, 0)))

    flops = 2 * b_pad * h * w * w_out + 8 * b_pad * h_out * w_out
    bytes_accessed = (b_pad * h * w + b_pad * h_out * w_out) * 4 + w * w_out * 4

    out_flat = pl.pallas_call(
        functools.partial(_up2x_kernel, h_in=h),
        out_shape=jax.ShapeDtypeStruct((b_pad, h_out, w_out), x.dtype),
        grid_spec=pltpu.PrefetchScalarGridSpec(
            num_scalar_prefetch=0,
            grid=(num_blocks,),
            in_specs=[
                pl.BlockSpec((bc, h, w), lambda i: (i, 0, 0)),
                pl.BlockSpec((w, w_out), lambda i: (0, 0)),
            ],
            out_specs=pl.BlockSpec((bc, h_out, w_out), lambda i: (i, 0, 0)),
        ),
        compiler_params=pltpu.CompilerParams(
            dimension_semantics=("parallel",),
            vmem_limit_bytes=_VMEM_LIMIT),
        cost_estimate=pl.CostEstimate(
            flops=int(flops), transcendentals=0,
            bytes_accessed=int(bytes_accessed)),
    )(x_flat, a_w_t)

    if b_pad != b:
        out_flat = out_flat[:b]
    return out_flat.reshape(n, c, h_out, w_out)


# speedup vs baseline: 1.9271x; 1.5335x over previous
import functools

import jax
import jax.numpy as jnp
from jax.experimental import pallas as pl
from jax.experimental.pallas import tpu as pltpu

_VMEM_LIMIT = 64 * 1024 * 1024


def _interp_matrix_t(n_in: int, n_out: int) -> jnp.ndarray:
    if n_out == 1 or n_in == 1:
        src = jnp.zeros((n_out,), dtype=jnp.float32)
    else:
        src = jnp.arange(n_out, dtype=jnp.float32) * ((n_in - 1) / (n_out - 1))
    i0 = jnp.clip(jnp.floor(src).astype(jnp.int32), 0, n_in - 1)
    i1 = jnp.clip(i0 + 1, 0, n_in - 1)
    frac = src - i0.astype(jnp.float32)
    m0 = jax.nn.one_hot(i0, n_in, dtype=jnp.float32) * (1.0 - frac)[:, None]
    m1 = jax.nn.one_hot(i1, n_in, dtype=jnp.float32) * frac[:, None]
    return (m0 + m1).T


def _up2x_kernel(x_ref, aht_ref, awt_ref, o_ref, *, h_in: int):
    del h_in
    bc, h, w = x_ref.shape
    h_out = aht_ref.shape[1]
    w_out = awt_ref.shape[1]

    xt = jnp.swapaxes(x_ref[...], 1, 2)
    v = jnp.dot(
        xt.reshape(bc * w, h), aht_ref[...],
        preferred_element_type=jnp.float32,
    ).reshape(bc, w, h_out)

    vt = jnp.swapaxes(v, 1, 2)
    out = jnp.dot(
        vt.reshape(bc * h_out, w), awt_ref[...],
        preferred_element_type=jnp.float32,
    )
    o_ref[...] = out.reshape(bc, h_out, w_out)


def kernel(x: jnp.ndarray) -> jnp.ndarray:
    n, c, h, w = x.shape
    h_out, w_out = 2 * h, 2 * w
    b = n * c

    a_h_t = _interp_matrix_t(h, h_out)
    a_w_t = _interp_matrix_t(w, w_out)

    bc = 64
    bc = max(1, min(bc, b))
    num_blocks = -(-b // bc)
    b_pad = num_blocks * bc

    x_flat = x.reshape(b, h, w)
    if b_pad != b:
        x_flat = jnp.pad(x_flat, ((0, b_pad - b), (0, 0), (0, 0)))

    flops = 2 * b_pad * h * w * w_out + 8 * b_pad * h_out * w_out
    bytes_accessed = (b_pad * h * w + b_pad * h_out * w_out) * 4 + w * w_out * 4

    out_flat = pl.pallas_call(
        functools.partial(_up2x_kernel, h_in=h),
        out_shape=jax.ShapeDtypeStruct((b_pad, h_out, w_out), x.dtype),
        grid_spec=pltpu.PrefetchScalarGridSpec(
            num_scalar_prefetch=0,
            grid=(num_blocks,),
            in_specs=[
                pl.BlockSpec((bc, h, w), lambda i: (i, 0, 0)),
                pl.BlockSpec((h, h_out), lambda i: (0, 0)),
                pl.BlockSpec((w, w_out), lambda i: (0, 0)),
            ],
            out_specs=pl.BlockSpec((bc, h_out, w_out), lambda i: (i, 0, 0)),
        ),
        compiler_params=pltpu.CompilerParams(
            dimension_semantics=("parallel",),
            vmem_limit_bytes=_VMEM_LIMIT),
        cost_estimate=pl.CostEstimate(
            flops=int(flops), transcendentals=0,
            bytes_accessed=int(bytes_accessed)),
    )(x_flat, a_h_t, a_w_t)

    if b_pad != b:
        out_flat = out_flat[:b]
    return out_flat.reshape(n, c, h_out, w_out)


# speedup vs baseline: 2.1115x; 1.0957x over previous
import functools

import jax
import jax.numpy as jnp
from jax.experimental import pallas as pl
from jax.experimental.pallas import tpu as pltpu

_VMEM_LIMIT = 64 * 1024 * 1024


def _interp_matrix_t(n_in: int, n_out: int) -> jnp.ndarray:
    if n_out == 1 or n_in == 1:
        src = jnp.zeros((n_out,), dtype=jnp.float32)
    else:
        src = jnp.arange(n_out, dtype=jnp.float32) * ((n_in - 1) / (n_out - 1))
    i0 = jnp.clip(jnp.floor(src).astype(jnp.int32), 0, n_in - 1)
    i1 = jnp.clip(i0 + 1, 0, n_in - 1)
    frac = src - i0.astype(jnp.float32)
    m0 = jax.nn.one_hot(i0, n_in, dtype=jnp.float32) * (1.0 - frac)[:, None]
    m1 = jax.nn.one_hot(i1, n_in, dtype=jnp.float32) * frac[:, None]
    return (m0 + m1).T


def _up2x_kernel(x_ref, aht_ref, awt_ref, o_ref, *, h_in: int):
    del h_in
    bc, h, w = x_ref.shape
    h_out = aht_ref.shape[1]
    w_out = awt_ref.shape[1]

    xt = jnp.swapaxes(x_ref[...], 1, 2)
    v = jnp.dot(
        xt.reshape(bc * w, h), aht_ref[...],
        preferred_element_type=jnp.float32,
    ).reshape(bc, w, h_out)

    vt = jnp.swapaxes(v, 1, 2)
    out = jnp.dot(
        vt.reshape(bc * h_out, w), awt_ref[...],
        preferred_element_type=jnp.float32,
    )
    o_ref[...] = out.reshape(bc, h_out, w_out)


def kernel(x: jnp.ndarray) -> jnp.ndarray:
    n, c, h, w = x.shape
    h_out, w_out = 2 * h, 2 * w
    b = n * c

    a_h_t = _interp_matrix_t(h, h_out)
    a_w_t = _interp_matrix_t(w, w_out)

    bc = 128
    bc = max(1, min(bc, b))
    num_blocks = -(-b // bc)
    b_pad = num_blocks * bc

    x_flat = x.reshape(b, h, w)
    if b_pad != b:
        x_flat = jnp.pad(x_flat, ((0, b_pad - b), (0, 0), (0, 0)))

    flops = 2 * b_pad * h * w * w_out + 8 * b_pad * h_out * w_out
    bytes_accessed = (b_pad * h * w + b_pad * h_out * w_out) * 4 + w * w_out * 4

    out_flat = pl.pallas_call(
        functools.partial(_up2x_kernel, h_in=h),
        out_shape=jax.ShapeDtypeStruct((b_pad, h_out, w_out), x.dtype),
        grid_spec=pltpu.PrefetchScalarGridSpec(
            num_scalar_prefetch=0,
            grid=(num_blocks,),
            in_specs=[
                pl.BlockSpec((bc, h, w), lambda i: (i, 0, 0)),
                pl.BlockSpec((h, h_out), lambda i: (0, 0)),
                pl.BlockSpec((w, w_out), lambda i: (0, 0)),
            ],
            out_specs=pl.BlockSpec((bc, h_out, w_out), lambda i: (i, 0, 0)),
        ),
        compiler_params=pltpu.CompilerParams(
            dimension_semantics=("parallel",),
            vmem_limit_bytes=_VMEM_LIMIT),
        cost_estimate=pl.CostEstimate(
            flops=int(flops), transcendentals=0,
            bytes_accessed=int(bytes_accessed)),
    )(x_flat, a_h_t, a_w_t)

    if b_pad != b:
        out_flat = out_flat[:b]
    return out_flat.reshape(n, c, h_out, w_out)
